# baseline (speedup 1.0000x reference)
import functools

import jax
import jax.numpy as jnp
from jax import lax
from jax.experimental import pallas as pl
from jax.experimental.pallas import tpu as pltpu
from jax.experimental.pallas import tpu_sc as plsc

_N = 10000
_E = 320000
_NC = 2
_NS = 16
_NW = _NC * _NS
_CH = 128
_NCHUNK = 80
_EPW = _CH * _NCHUNK
_EPAD = _NW * _EPW
_NPAD = _N + 16
_RPT = 624
_REM = _N - _NS * _RPT


def _make_sc_agg(d, gather):
  mesh = plsc.VectorSubcoreMesh(core_axis_name="c", subcore_axis_name="s")
  out_type = [jax.ShapeDtypeStruct((_NC, _N, d), jnp.float32)]
  scratch = [
      pltpu.VMEM_SHARED((_NPAD, d), jnp.float32),
      pltpu.VMEM((_CH, d), jnp.float32),
  ]
  if gather:
    scratch += [
        pltpu.VMEM((_CH, d), jnp.float32),
        pltpu.VMEM((_CH,), jnp.int32),
        pltpu.VMEM((_CH,), jnp.int32),
        pltpu.VMEM((_CH,), jnp.int32),
        pltpu.VMEM((_CH,), jnp.int32),
        pltpu.SemaphoreType.DMA,
        pltpu.SemaphoreType.DMA,
    ]
  else:
    scratch.append(pltpu.VMEM((_NCHUNK, _CH), jnp.int32))

  def body(*refs):
    if gather:
      (h, srcp, dstp, zrows, out_acc,
       acc_sh, bufA, bufB, isA, idA, isB, idB, semA, semB) = refs
    else:
      (ones_hbm, dstp, zrows, out_acc, acc_sh, bufA, idx_d) = refs

    c = lax.axis_index("c")
    s = lax.axis_index("s")
    wid = s * _NC + c
    base = wid * _NCHUNK
    row0 = s * _RPT

    pltpu.sync_copy(zrows, acc_sh.at[pl.ds(row0, _RPT)])

    @pl.when(s == _NS - 1)
    def _():
      pltpu.sync_copy(zrows.at[pl.ds(0, _NPAD - _NS * _RPT)],
                      acc_sh.at[pl.ds(_NS * _RPT, _NPAD - _NS * _RPT)])

    if gather:
      pltpu.sync_copy(srcp.at[base], isA)
      pltpu.sync_copy(dstp.at[base], idA)
    else:
      pltpu.sync_copy(dstp.at[pl.ds(base, _NCHUNK)], idx_d)
      pltpu.sync_copy(ones_hbm, bufA)
    plsc.subcore_barrier()

    if gather:
      pltpu.async_copy(h.at[isA], bufA, semA)

      def step(g2, carry):
        a = base + 2 * g2
        pltpu.sync_copy(srcp.at[a + 1], isB)
        pltpu.sync_copy(dstp.at[a + 1], idB)
        pltpu.async_copy(h.at[isB], bufB, semB)
        pltpu.make_async_copy(h.at[isA], bufA, semA).wait()
        pltpu.sync_copy(bufA, acc_sh.at[idA], add=True)

        @pl.when(g2 < _NCHUNK // 2 - 1)
        def _():
          pltpu.sync_copy(srcp.at[a + 2], isA)
          pltpu.sync_copy(dstp.at[a + 2], idA)
          pltpu.async_copy(h.at[isA], bufA, semA)

        pltpu.make_async_copy(h.at[isB], bufB, semB).wait()
        pltpu.sync_copy(bufB, acc_sh.at[idB], add=True)
        return carry

      lax.fori_loop(0, _NCHUNK // 2, step, 0)
    else:
      def step(g, carry):
        pltpu.sync_copy(bufA, acc_sh.at[idx_d.at[g]], add=True)
        return carry

      lax.fori_loop(0, _NCHUNK, step, 0)
    plsc.subcore_barrier()

    pltpu.sync_copy(acc_sh.at[pl.ds(row0, _RPT)],
                    out_acc.at[c].at[pl.ds(row0, _RPT)])

    @pl.when(s == _NS - 1)
    def _():
      pltpu.sync_copy(acc_sh.at[pl.ds(_NS * _RPT, _REM)],
                      out_acc.at[c].at[pl.ds(_NS * _RPT, _REM)])

  k = pl.kernel(body, out_type=out_type, mesh=mesh, scratch_types=scratch,
                compiler_params=pltpu.CompilerParams(needs_layout_passes=False))
  return lambda *a: k(*a)[0]


_sc_agg128 = _make_sc_agg(128, True)
_sc_deg128 = _make_sc_agg(128, False)

_BN = 1024


def _mean(acc_ref, deg_ref):
  acc = acc_ref[0] + acc_ref[1]
  d = deg_ref[0] + deg_ref[1]
  deg = d[:, 0]
  inv = 1.0 / jnp.maximum(deg, 1.0)
  return acc * inv[:, None]


def _tc1_body(x_ref, acc_ref, deg_ref, ws_ref, wn_ref, b_ref, o_ref):
  agg = _mean(acc_ref, deg_ref)
  h = (jnp.dot(x_ref[...], ws_ref[...], preferred_element_type=jnp.float32)
       + jnp.dot(agg, wn_ref[...], preferred_element_type=jnp.float32)
       + b_ref[...])
  o_ref[...] = jnp.maximum(h, 0.0)


def _tc3_body(x_ref, acc_ref, deg_ref, ws_ref, wn_ref, b_ref, o_ref):
  agg = _mean(acc_ref, deg_ref)
  z = (jnp.dot(x_ref[...], ws_ref[...], preferred_element_type=jnp.float32)
       + jnp.dot(agg, wn_ref[...], preferred_element_type=jnp.float32)
       + b_ref[...])
  col = lax.broadcasted_iota(jnp.int32, z.shape, 1)
  valid = col < 47
  zm = jnp.where(valid, z, -1e30)
  m = jnp.max(zm, axis=1, keepdims=True)
  e = jnp.where(valid, jnp.exp(zm - m), 0.0)
  lse = jnp.log(jnp.sum(e, axis=1, keepdims=True)) + m
  o_ref[...] = z - lse


def _row_spec(d):
  return pl.BlockSpec((_BN, d), lambda i: (i, 0))


def _full_spec(r, c):
  return pl.BlockSpec((r, c), lambda i: (0, 0))


def _acc_spec(d):
  return pl.BlockSpec((_NC, _BN, d), lambda i: (0, i, 0))


_GRID = (pl.cdiv(_N, _BN),)

_tc1 = pl.pallas_call(
    _tc1_body,
    grid=_GRID,
    in_specs=[_row_spec(128), _acc_spec(128), _acc_spec(128),
              _full_spec(128, 128), _full_spec(128, 128), _full_spec(1, 128)],
    out_specs=_row_spec(128),
    out_shape=jax.ShapeDtypeStruct((_N, 128), jnp.float32),
)

_tc3 = pl.pallas_call(
    _tc3_body,
    grid=_GRID,
    in_specs=[_row_spec(128), _acc_spec(128), _acc_spec(128),
              _full_spec(128, 128), _full_spec(128, 128), _full_spec(1, 128)],
    out_specs=_row_spec(128),
    out_shape=jax.ShapeDtypeStruct((_N, 128), jnp.float32),
)


def kernel(x, edge_index, Ws1, Wn1, b1, Ws2, Wn2, b2, Ws3, Wn3, b3):
  src = edge_index[0].astype(jnp.int32)
  dst = edge_index[1].astype(jnp.int32)
  rpw = _E // _NW
  pad_pw = _EPW - rpw
  pad_dst = (_N + jnp.arange(_NW, dtype=jnp.int32) // _NC)[:, None]
  srcp = jnp.concatenate(
      [src.reshape(_NW, rpw), jnp.zeros((_NW, pad_pw), jnp.int32)], axis=1
      ).reshape(_NW * _NCHUNK, _CH)
  dstp = jnp.concatenate(
      [dst.reshape(_NW, rpw),
       jnp.broadcast_to(pad_dst, (_NW, pad_pw)).astype(jnp.int32)], axis=1
      ).reshape(_NW * _NCHUNK, _CH)
  z128 = jnp.zeros((_RPT, 128), jnp.float32)
  ones_blk = jnp.ones((_CH, 128), jnp.float32)

  degacc = _sc_deg128(ones_blk, dstp, z128)
  accx = _sc_agg128(x, srcp, dstp, z128)
  h1 = _tc1(x, accx, degacc, Ws1, Wn1, b1.reshape(1, -1))

  acch = _sc_agg128(h1, srcp, dstp, z128)
  h2 = _tc1(h1, acch, degacc, Ws2, Wn2, b2.reshape(1, -1))

  acc2 = _sc_agg128(h2, srcp, dstp, z128)
  ws3p = jnp.zeros((128, 128), jnp.float32).at[:, :47].set(Ws3)
  wn3p = jnp.zeros((128, 128), jnp.float32).at[:, :47].set(Wn3)
  b3p = jnp.zeros((1, 128), jnp.float32).at[0, :47].set(b3)
  z = _tc3(h2, acc2, degacc, ws3p, wn3p, b3p)
  return z[:, :47]


# speedup vs baseline: 2.2850x; 2.2850x over previous
import functools

import jax
import jax.numpy as jnp
from jax import lax
from jax.experimental import pallas as pl
from jax.experimental.pallas import tpu as pltpu
from jax.experimental.pallas import tpu_sc as plsc

_N = 10000
_E = 320000
_NC = 2
_NS = 16
_NW = _NC * _NS
_CH = 128
_NCHUNK = 80
_EPW = _CH * _NCHUNK
_EPAD = _NW * _EPW
_NPAD = _N + 16
_RPT = 624
_REM = _N - _NS * _RPT


def _make_sc_agg(d, gather):
  mesh = plsc.VectorSubcoreMesh(core_axis_name="c", subcore_axis_name="s")
  out_type = [jax.ShapeDtypeStruct((_NC, _N, d), jnp.float32)]
  scratch = [
      pltpu.VMEM_SHARED((_NPAD, d), jnp.float32),
      pltpu.VMEM((_CH, d), jnp.float32),
  ]
  if gather:
    scratch += [
        pltpu.VMEM((_CH, d), jnp.float32),
        pltpu.VMEM((_CH,), jnp.int32),
        pltpu.VMEM((_CH,), jnp.int32),
        pltpu.VMEM((_CH,), jnp.int32),
        pltpu.VMEM((_CH,), jnp.int32),
        pltpu.SemaphoreType.DMA,
        pltpu.SemaphoreType.DMA,
    ]
  else:
    scratch.append(pltpu.VMEM((_CH,), jnp.int32))

  def body(*refs):
    if gather:
      (h, srcp, dstp, zrows, out_acc,
       acc_sh, bufA, bufB, isA, idA, isB, idB, semA, semB) = refs
    else:
      (ones_hbm, dstp, zrows, out_acc, acc_sh, bufA, idA) = refs

    c = lax.axis_index("c")
    s = lax.axis_index("s")
    wid = s * _NC + c
    row0 = s * _RPT

    pltpu.sync_copy(zrows, acc_sh.at[pl.ds(row0, _RPT)])

    @pl.when(s == _NS - 1)
    def _():
      pltpu.sync_copy(zrows.at[pl.ds(0, _NPAD - _NS * _RPT)],
                      acc_sh.at[pl.ds(_NS * _RPT, _NPAD - _NS * _RPT)])

    if gather:
      pltpu.sync_copy(srcp.at[wid], isA)
      pltpu.sync_copy(dstp.at[wid], idA)
    else:
      pltpu.sync_copy(ones_hbm, bufA)
    plsc.subcore_barrier()

    if gather:
      pltpu.async_copy(h.at[isA], bufA, semA)

      def step(g2, carry):
        rb = (2 * g2 + 1) * _NW + wid
        pltpu.sync_copy(srcp.at[rb], isB)
        pltpu.sync_copy(dstp.at[rb], idB)
        pltpu.async_copy(h.at[isB], bufB, semB)
        pltpu.make_async_copy(h.at[isA], bufA, semA).wait()
        pltpu.sync_copy(bufA, acc_sh.at[idA], add=True)

        @pl.when(g2 < _NCHUNK // 2 - 1)
        def _():
          ra = (2 * g2 + 2) * _NW + wid
          pltpu.sync_copy(srcp.at[ra], isA)
          pltpu.sync_copy(dstp.at[ra], idA)
          pltpu.async_copy(h.at[isA], bufA, semA)

        pltpu.make_async_copy(h.at[isB], bufB, semB).wait()
        pltpu.sync_copy(bufB, acc_sh.at[idB], add=True)
        return carry

      lax.fori_loop(0, _NCHUNK // 2, step, 0)
    else:
      def step(g, carry):
        pltpu.sync_copy(dstp.at[g * _NW + wid], idA)
        pltpu.sync_copy(bufA, acc_sh.at[idA], add=True)
        return carry

      lax.fori_loop(0, _NCHUNK, step, 0)
    plsc.subcore_barrier()

    pltpu.sync_copy(acc_sh.at[pl.ds(row0, _RPT)],
                    out_acc.at[c].at[pl.ds(row0, _RPT)])

    @pl.when(s == _NS - 1)
    def _():
      pltpu.sync_copy(acc_sh.at[pl.ds(_NS * _RPT, _REM)],
                      out_acc.at[c].at[pl.ds(_NS * _RPT, _REM)])

  k = pl.kernel(body, out_type=out_type, mesh=mesh, scratch_types=scratch,
                compiler_params=pltpu.CompilerParams(needs_layout_passes=False))
  return lambda *a: k(*a)[0]


_sc_agg128 = _make_sc_agg(128, True)
_sc_deg128 = _make_sc_agg(128, False)

_BN = 1024


def _mean(acc_ref, deg_ref):
  acc = acc_ref[0] + acc_ref[1]
  d = deg_ref[0] + deg_ref[1]
  deg = d[:, 0]
  inv = 1.0 / jnp.maximum(deg, 1.0)
  return acc * inv[:, None]


def _tc1_body(x_ref, acc_ref, deg_ref, ws_ref, wn_ref, b_ref, o_ref):
  agg = _mean(acc_ref, deg_ref)
  h = (jnp.dot(x_ref[...], ws_ref[...], preferred_element_type=jnp.float32)
       + jnp.dot(agg, wn_ref[...], preferred_element_type=jnp.float32)
       + b_ref[...])
  o_ref[...] = jnp.maximum(h, 0.0)


def _tc3_body(x_ref, acc_ref, deg_ref, ws_ref, wn_ref, b_ref, o_ref):
  agg = _mean(acc_ref, deg_ref)
  z = (jnp.dot(x_ref[...], ws_ref[...], preferred_element_type=jnp.float32)
       + jnp.dot(agg, wn_ref[...], preferred_element_type=jnp.float32)
       + b_ref[...])
  col = lax.broadcasted_iota(jnp.int32, z.shape, 1)
  valid = col < 47
  zm = jnp.where(valid, z, -1e30)
  m = jnp.max(zm, axis=1, keepdims=True)
  e = jnp.where(valid, jnp.exp(zm - m), 0.0)
  lse = jnp.log(jnp.sum(e, axis=1, keepdims=True)) + m
  o_ref[...] = z - lse


def _row_spec(d):
  return pl.BlockSpec((_BN, d), lambda i: (i, 0))


def _full_spec(r, c):
  return pl.BlockSpec((r, c), lambda i: (0, 0))


def _acc_spec(d):
  return pl.BlockSpec((_NC, _BN, d), lambda i: (0, i, 0))


_GRID = (pl.cdiv(_N, _BN),)

_tc1 = pl.pallas_call(
    _tc1_body,
    grid=_GRID,
    in_specs=[_row_spec(128), _acc_spec(128), _acc_spec(128),
              _full_spec(128, 128), _full_spec(128, 128), _full_spec(1, 128)],
    out_specs=_row_spec(128),
    out_shape=jax.ShapeDtypeStruct((_N, 128), jnp.float32),
)

_tc3 = pl.pallas_call(
    _tc3_body,
    grid=_GRID,
    in_specs=[_row_spec(128), _acc_spec(128), _acc_spec(128),
              _full_spec(128, 128), _full_spec(128, 128), _full_spec(1, 128)],
    out_specs=_row_spec(128),
    out_shape=jax.ShapeDtypeStruct((_N, 128), jnp.float32),
)


def kernel(x, edge_index, Ws1, Wn1, b1, Ws2, Wn2, b2, Ws3, Wn3, b3):
  src = edge_index[0].astype(jnp.int32)
  dst = edge_index[1].astype(jnp.int32)
  pad = _EPAD - _E
  ar = jnp.arange(pad, dtype=jnp.int32)
  pad_src = (ar * 97) % _N
  pad_dst = _N + (ar // _CH) % 16
  srcp = jnp.concatenate([src, pad_src]).reshape(_NW * _NCHUNK, _CH)
  dstp = jnp.concatenate([dst, pad_dst]).reshape(_NW * _NCHUNK, _CH)
  z128 = jnp.zeros((_RPT, 128), jnp.float32)
  ones_blk = jnp.ones((_CH, 128), jnp.float32)

  degacc = _sc_deg128(ones_blk, dstp, z128)
  accx = _sc_agg128(x, srcp, dstp, z128)
  h1 = _tc1(x, accx, degacc, Ws1, Wn1, b1.reshape(1, -1))

  acch = _sc_agg128(h1, srcp, dstp, z128)
  h2 = _tc1(h1, acch, degacc, Ws2, Wn2, b2.reshape(1, -1))

  acc2 = _sc_agg128(h2, srcp, dstp, z128)
  ws3p = jnp.zeros((128, 128), jnp.float32).at[:, :47].set(Ws3)
  wn3p = jnp.zeros((128, 128), jnp.float32).at[:, :47].set(Wn3)
  b3p = jnp.zeros((1, 128), jnp.float32).at[0, :47].set(b3)
  z = _tc3(h2, acc2, degacc, ws3p, wn3p, b3p)
  return z[:, :47]
